# baseline (speedup 1.0000x reference)
import jax
import jax.numpy as jnp
from jax import lax
from jax.experimental import pallas as pl
from jax.experimental.pallas import tpu as pltpu
from jax.experimental.pallas import tpu_sc as plsc

N = 10000
FEAT = 128
HID = 64

NPAD = 10240
HALF = NPAD // 2
ACC_ROWS = HALF + 128
RPT_OUT = HALF // 16
RPT_Z = ACC_ROWS // 16
CHUNK = 128
BCH = 8

_MESH = plsc.VectorSubcoreMesh(
    core_axis_name="c", subcore_axis_name="s", num_cores=2, num_subcores=16
)


_SC_SCRATCH = [
    pltpu.VMEM((2, BCH, CHUNK), jnp.int32),
    pltpu.VMEM((2, BCH, CHUNK), jnp.int32),
    pltpu.VMEM((CHUNK, FEAT), jnp.float32),
    pltpu.VMEM((CHUNK, FEAT), jnp.float32),
    pltpu.VMEM((16, 16), jnp.int32),
    pltpu.VMEM_SHARED((ACC_ROWS, FEAT), jnp.float32),
    pltpu.SemaphoreType.DMA,
    pltpu.SemaphoreType.DMA,
    pltpu.SemaphoreType.DMA,
    pltpu.SemaphoreType.DMA,
    pltpu.SemaphoreType.DMA,
    pltpu.SemaphoreType.DMA,
]


def _my_batches(cnt_v, s):
    return cnt_v[s, :][0]


def _agg_kernel(cap_b):
    def body(h_hbm, src_hbm, dst_hbm, z_hbm, cnt_hbm, out_hbm,
             idx_s, idx_d, rows_a, rows_b, cnt_v, acc_sh,
             gs0, gs1, ss0, ss1, is0, is1):
        c = lax.axis_index("c")
        s = lax.axis_index("s")
        w = c * 16 + s
        pltpu.sync_copy(
            z_hbm.at[pl.ds(s * RPT_Z, RPT_Z)],
            acc_sh.at[pl.ds(s * RPT_Z, RPT_Z)],
        )
        pltpu.sync_copy(cnt_hbm.at[c], cnt_v)
        nb = _my_batches(cnt_v, s)

        @pl.when(nb > 0)
        def _():
            pltpu.sync_copy(src_hbm.at[w, 0], idx_s.at[0])
            pltpu.sync_copy(dst_hbm.at[w, 0], idx_d.at[0])

        plsc.subcore_barrier()

        bufs = (rows_a, rows_b)
        gsems = (gs0, gs1)
        ssems = (ss0, ss1)
        isems = (is0, is1)

        def outer(b, carry):
            p = lax.rem(b, 2)
            q = lax.rem(b + 1, 2)

            @pl.when(b + 1 < nb)
            def _():
                pltpu.async_copy(src_hbm.at[w, b + 1], idx_s.at[q], isems[0])
                pltpu.async_copy(dst_hbm.at[w, b + 1], idx_d.at[q], isems[1])

            @pl.when(b > 0)
            def _():
                pltpu.make_async_copy(src_hbm.at[w, b], idx_s.at[p], isems[0]).wait()
                pltpu.make_async_copy(dst_hbm.at[w, b], idx_d.at[p], isems[1]).wait()

            sc_desc = [None, None]
            g_desc = [None, None]
            g_desc[0] = pltpu.async_copy(
                h_hbm.at[idx_s.at[p, 0]], bufs[0], gsems[0]
            )
            for j in range(BCH):
                cur = j % 2
                nxt = (j + 1) % 2
                if j + 1 < BCH:
                    if sc_desc[nxt] is not None:
                        sc_desc[nxt].wait()
                        sc_desc[nxt] = None
                    g_desc[nxt] = pltpu.async_copy(
                        h_hbm.at[idx_s.at[p, j + 1]], bufs[nxt], gsems[nxt]
                    )
                g_desc[cur].wait()
                sc_desc[cur] = pltpu.async_copy(
                    bufs[cur], acc_sh.at[idx_d.at[p, j]], ssems[cur], add=True
                )
            for d in sc_desc:
                if d is not None:
                    d.wait()
            return carry

        lax.fori_loop(0, nb, outer, 0)
        plsc.subcore_barrier()
        pltpu.sync_copy(
            acc_sh.at[pl.ds(s * RPT_OUT, RPT_OUT)],
            out_hbm.at[c, pl.ds(s * RPT_OUT, RPT_OUT)],
        )

    return pl.kernel(
        body,
        out_type=jax.ShapeDtypeStruct((2, HALF, FEAT), jnp.float32),
        mesh=_MESH,
        scratch_types=list(_SC_SCRATCH),
    )


def _deg_kernel(cap_b):
    def body(dst_hbm, z_hbm, cnt_hbm, out_hbm,
             idx_s, idx_d, rows_a, rows_b, cnt_v, acc_sh,
             gs0, gs1, ss0, ss1, is0, is1):
        c = lax.axis_index("c")
        s = lax.axis_index("s")
        w = c * 16 + s
        ones16 = jnp.ones((16,), jnp.float32)

        def fill(i, carry):
            for k in range(FEAT // 16):
                rows_a[i, pl.ds(k * 16, 16)] = ones16
            return carry

        lax.fori_loop(0, CHUNK, fill, 0)
        pltpu.sync_copy(
            z_hbm.at[pl.ds(s * RPT_Z, RPT_Z)],
            acc_sh.at[pl.ds(s * RPT_Z, RPT_Z)],
        )
        pltpu.sync_copy(cnt_hbm.at[c], cnt_v)
        nb = _my_batches(cnt_v, s)

        @pl.when(nb > 0)
        def _():
            pltpu.sync_copy(dst_hbm.at[w, 0], idx_d.at[0])

        plsc.subcore_barrier()

        ssems = (ss0, ss1)
        isems = (is0, is1)

        def outer(b, carry):
            p = lax.rem(b, 2)
            q = lax.rem(b + 1, 2)

            @pl.when(b + 1 < nb)
            def _():
                pltpu.async_copy(dst_hbm.at[w, b + 1], idx_d.at[q], isems[1])

            @pl.when(b > 0)
            def _():
                pltpu.make_async_copy(dst_hbm.at[w, b], idx_d.at[p], isems[1]).wait()

            sc_desc = [None, None]
            for j in range(BCH):
                cur = j % 2
                if sc_desc[cur] is not None:
                    sc_desc[cur].wait()
                sc_desc[cur] = pltpu.async_copy(
                    rows_a, acc_sh.at[idx_d.at[p, j]], ssems[cur], add=True
                )
            for d in sc_desc:
                if d is not None:
                    d.wait()
            return carry

        lax.fori_loop(0, nb, outer, 0)
        plsc.subcore_barrier()
        pltpu.sync_copy(
            acc_sh.at[pl.ds(s * RPT_OUT, RPT_OUT)],
            out_hbm.at[c, pl.ds(s * RPT_OUT, RPT_OUT)],
        )

    return pl.kernel(
        body,
        out_type=jax.ShapeDtypeStruct((2, HALF, FEAT), jnp.float32),
        mesh=_MESH,
        scratch_types=list(_SC_SCRATCH),
    )


_BR = 1280


def _dinv_block(deg_ref):
    return lax.rsqrt(jnp.maximum(deg_ref[:, :1], 1.0))


def _first_mm(x, w1, deg):
    def body(x_ref, w_ref, deg_ref, o_ref):
        dinv = _dinv_block(deg_ref)
        t = jnp.dot(x_ref[...], w_ref[...], preferred_element_type=jnp.float32)
        o_ref[...] = jnp.concatenate(
            [t * dinv, jnp.zeros((_BR, FEAT - HID), jnp.float32)], axis=1
        )

    return pl.pallas_call(
        body,
        grid=(NPAD // _BR,),
        in_specs=[
            pl.BlockSpec((_BR, FEAT), lambda i: (i, 0)),
            pl.BlockSpec((FEAT, HID), lambda i: (0, 0)),
            pl.BlockSpec((_BR, FEAT), lambda i: (i, 0)),
        ],
        out_specs=pl.BlockSpec((_BR, FEAT), lambda i: (i, 0)),
        out_shape=jax.ShapeDtypeStruct((NPAD, FEAT), jnp.float32),
    )(x, w1, deg)


def _mid_layer(agg, deg, b2d, wn):
    fo = wn.shape[1]

    def body(p_ref, deg_ref, b_ref, w_ref, o_ref):
        dinv = _dinv_block(deg_ref)
        u = jax.nn.relu(p_ref[:, :HID] * dinv + b_ref[0:1, :])
        t = jnp.dot(u, w_ref[...], preferred_element_type=jnp.float32) * dinv
        if fo == FEAT:
            o_ref[...] = t
        else:
            o_ref[...] = jnp.concatenate(
                [t, jnp.zeros((_BR, FEAT - fo), jnp.float32)], axis=1
            )

    return pl.pallas_call(
        body,
        grid=(NPAD // _BR,),
        in_specs=[
            pl.BlockSpec((_BR, FEAT), lambda i: (i, 0)),
            pl.BlockSpec((_BR, FEAT), lambda i: (i, 0)),
            pl.BlockSpec((8, HID), lambda i: (0, 0)),
            pl.BlockSpec((HID, fo), lambda i: (0, 0)),
        ],
        out_specs=pl.BlockSpec((_BR, FEAT), lambda i: (i, 0)),
        out_shape=jax.ShapeDtypeStruct((NPAD, FEAT), jnp.float32),
    )(agg, deg, b2d, wn)


def _hidden_layer(agg, deg, b2d, wa, ws):
    def body(p_ref, deg_ref, b_ref, wa_ref, ws_ref, o_ref):
        dinv = _dinv_block(deg_ref)
        u = jax.nn.relu(p_ref[:, :HID] * dinv + b_ref[0:1, :])
        ta = jnp.dot(u, wa_ref[...], preferred_element_type=jnp.float32)
        ts = jnp.dot(u, ws_ref[...], preferred_element_type=jnp.float32)
        o_ref[...] = jnp.concatenate([ta, ts], axis=1) * dinv

    return pl.pallas_call(
        body,
        grid=(NPAD // _BR,),
        in_specs=[
            pl.BlockSpec((_BR, FEAT), lambda i: (i, 0)),
            pl.BlockSpec((_BR, FEAT), lambda i: (i, 0)),
            pl.BlockSpec((8, HID), lambda i: (0, 0)),
            pl.BlockSpec((HID, HID), lambda i: (0, 0)),
            pl.BlockSpec((HID, HID), lambda i: (0, 0)),
        ],
        out_specs=pl.BlockSpec((_BR, FEAT), lambda i: (i, 0)),
        out_shape=jax.ShapeDtypeStruct((NPAD, FEAT), jnp.float32),
    )(agg, deg, b2d, wa, ws)


def _final_act(agg, deg, b2d, lo, width):
    def body(p_ref, deg_ref, b_ref, o_ref):
        dinv = _dinv_block(deg_ref)
        o_ref[...] = jax.nn.relu(
            p_ref[:, lo:lo + width] * dinv + b_ref[0:1, :]
        )

    return pl.pallas_call(
        body,
        grid=(NPAD // _BR,),
        in_specs=[
            pl.BlockSpec((_BR, FEAT), lambda i: (i, 0)),
            pl.BlockSpec((_BR, FEAT), lambda i: (i, 0)),
            pl.BlockSpec((8, width), lambda i: (0, 0)),
        ],
        out_specs=pl.BlockSpec((_BR, width), lambda i: (i, 0)),
        out_shape=jax.ShapeDtypeStruct((NPAD, width), jnp.float32),
    )(agg, deg, b2d)


_BS = 1024


def _struct_mm(s):
    def body(a_ref, b_ref, o_ref):
        o_ref[...] = lax.dot_general(
            a_ref[...], b_ref[...], (((1,), (1,)), ((), ())),
            preferred_element_type=jnp.float32,
        )

    nb = pl.cdiv(N, _BS)
    return pl.pallas_call(
        body,
        grid=(nb, nb),
        in_specs=[
            pl.BlockSpec((_BS, HID), lambda i, j: (i, 0)),
            pl.BlockSpec((_BS, HID), lambda i, j: (j, 0)),
        ],
        out_specs=pl.BlockSpec((_BS, _BS), lambda i, j: (i, j)),
        out_shape=jax.ShapeDtypeStruct((N, N), jnp.float32),
    )(s, s)


def kernel(x, edge_index, W1e, b1e, W2e, b2e, W1a, b1a, W2a, b2a, W1s, b1s):
    e = edge_index.shape[1]
    ea = e + N
    chunks_total = (ea + CHUNK - 1) // CHUNK
    cap_b = (((chunks_total + 15) // 16) + BCH - 1) // BCH
    cap_total = 32 * cap_b * BCH * CHUNK

    loop = jnp.arange(N, dtype=jnp.int32)
    src = jnp.concatenate([edge_index[0].astype(jnp.int32), loop])
    dst = jnp.concatenate([edge_index[1].astype(jnp.int32), loop])

    half = (dst >= HALF).astype(jnp.int32)
    r1 = jnp.cumsum(half)
    r0 = jnp.cumsum(1 - half)
    q = jnp.where(half == 1, r1, r0) - 1
    chunk = q // CHUNK
    lane = q % CHUNK
    tile = chunk % 16
    lc = chunk // 16
    b = lc // BCH
    j = lc % BCH
    w = half * 16 + tile
    flat = ((w * cap_b + b) * BCH + j) * CHUNK + lane
    dstl = dst - half * HALF
    ar = jnp.arange(cap_total, dtype=jnp.int32)
    trash_s = N + (ar % (NPAD - N))
    trash_d = HALF + (ar % (ACC_ROWS - HALF))
    src_p = trash_s.at[flat].set(src).reshape(32, cap_b, BCH, CHUNK)
    dst_p = trash_d.at[flat].set(dstl).reshape(32, cap_b, BCH, CHUNK)

    n_c = jnp.stack([r0[-1], r1[-1]])
    kc = (n_c + CHUNK - 1) // CHUNK
    t16 = jnp.arange(16, dtype=jnp.int32)
    chunks_t = jnp.maximum(0, (kc[:, None] - t16[None, :] + 15) // 16)
    nb_t = (chunks_t + BCH - 1) // BCH
    cnt = jnp.broadcast_to(nb_t[:, :, None], (2, 16, 16)).astype(jnp.int32)

    x_pad = jnp.zeros((NPAD, FEAT), jnp.float32).at[:N].set(x)
    b1e2 = jnp.broadcast_to(b1e, (8, HID))
    b2e2 = jnp.broadcast_to(b2e, (8, HID))
    b1a2 = jnp.broadcast_to(b1a, (8, HID))
    b2a2 = jnp.broadcast_to(b2a, (8, FEAT))
    b1s2 = jnp.broadcast_to(b1s, (8, HID))

    z128 = jnp.zeros((ACC_ROWS, FEAT), jnp.float32)
    agg0 = _agg_kernel(cap_b)
    agg = lambda h: agg0(h, src_p, dst_p, z128, cnt).reshape(NPAD, FEAT)
    deg = _deg_kernel(cap_b)(dst_p, z128, cnt).reshape(NPAD, FEAT)

    t1 = _first_mm(x_pad, W1e, deg)
    p1 = agg(t1)
    t2 = _mid_layer(p1, deg, b1e2, W2e)
    p2 = agg(t2)
    t35 = _hidden_layer(p2, deg, b2e2, W1a, W1s)
    p35 = agg(t35)
    t4 = _mid_layer(p35, deg, b1a2, W2a)
    p4 = agg(t4)
    x_hat = _final_act(p4, deg, b2a2, 0, FEAT)
    s = _final_act(p35, deg, b1s2, HID, HID)

    struct = _struct_mm(s)
    return (struct, x_hat[:N])


# speedup vs baseline: 3.1830x; 3.1830x over previous
import jax
import jax.numpy as jnp
from jax import lax
from jax.experimental import pallas as pl
from jax.experimental.pallas import tpu as pltpu
from jax.experimental.pallas import tpu_sc as plsc

N = 10000
FEAT = 128
HID = 64

NPAD = 10112
ROWS_PER_TILE = NPAD // 16
NT = 16
CHUNK = 128
BCH = 8

_MESH = plsc.VectorSubcoreMesh(
    core_axis_name="c", subcore_axis_name="s", num_cores=1, num_subcores=16
)


_SC_SCRATCH = [
    pltpu.VMEM((2, BCH, CHUNK), jnp.int32),
    pltpu.VMEM((2, BCH, CHUNK), jnp.int32),
    pltpu.VMEM((CHUNK, FEAT), jnp.float32),
    pltpu.VMEM((CHUNK, FEAT), jnp.float32),
    pltpu.VMEM_SHARED((NPAD, FEAT), jnp.float32),
    pltpu.SemaphoreType.DMA,
    pltpu.SemaphoreType.DMA,
    pltpu.SemaphoreType.DMA,
    pltpu.SemaphoreType.DMA,
    pltpu.SemaphoreType.DMA,
    pltpu.SemaphoreType.DMA,
]


def _agg_kernel(nbatch):
    def body(h_hbm, src_hbm, dst_hbm, z_hbm, out_hbm,
             idx_s, idx_d, rows_a, rows_b, acc_sh,
             gs0, gs1, ss0, ss1, is0, is1):
        s = lax.axis_index("s")
        pltpu.sync_copy(
            z_hbm.at[pl.ds(s * ROWS_PER_TILE, ROWS_PER_TILE)],
            acc_sh.at[pl.ds(s * ROWS_PER_TILE, ROWS_PER_TILE)],
        )
        pltpu.sync_copy(src_hbm.at[s, 0], idx_s.at[0])
        pltpu.sync_copy(dst_hbm.at[s, 0], idx_d.at[0])
        plsc.subcore_barrier()

        bufs = (rows_a, rows_b)
        gsems = (gs0, gs1)
        ssems = (ss0, ss1)
        isems = (is0, is1)

        def outer(b, carry):
            p = lax.rem(b, 2)
            q = lax.rem(b + 1, 2)

            @pl.when(b + 1 < nbatch)
            def _():
                pltpu.async_copy(src_hbm.at[s, b + 1], idx_s.at[q], isems[0])
                pltpu.async_copy(dst_hbm.at[s, b + 1], idx_d.at[q], isems[1])

            @pl.when(b > 0)
            def _():
                pltpu.make_async_copy(src_hbm.at[s, b], idx_s.at[p], isems[0]).wait()
                pltpu.make_async_copy(dst_hbm.at[s, b], idx_d.at[p], isems[1]).wait()

            sc_desc = [None, None]
            g_desc = [None, None]
            g_desc[0] = pltpu.async_copy(
                h_hbm.at[idx_s.at[p, 0]], bufs[0], gsems[0]
            )
            for j in range(BCH):
                cur = j % 2
                nxt = (j + 1) % 2
                if j + 1 < BCH:
                    if sc_desc[nxt] is not None:
                        sc_desc[nxt].wait()
                        sc_desc[nxt] = None
                    g_desc[nxt] = pltpu.async_copy(
                        h_hbm.at[idx_s.at[p, j + 1]], bufs[nxt], gsems[nxt]
                    )
                g_desc[cur].wait()
                sc_desc[cur] = pltpu.async_copy(
                    bufs[cur], acc_sh.at[idx_d.at[p, j]], ssems[cur], add=True
                )
            for d in sc_desc:
                if d is not None:
                    d.wait()
            return carry

        lax.fori_loop(0, nbatch, outer, 0)
        plsc.subcore_barrier()
        pltpu.sync_copy(
            acc_sh.at[pl.ds(s * ROWS_PER_TILE, ROWS_PER_TILE)],
            out_hbm.at[pl.ds(s * ROWS_PER_TILE, ROWS_PER_TILE)],
        )

    return pl.kernel(
        body,
        out_type=jax.ShapeDtypeStruct((NPAD, FEAT), jnp.float32),
        mesh=_MESH,
        scratch_types=list(_SC_SCRATCH),
    )


def _deg_kernel(nbatch):
    def body(dst_hbm, z_hbm, out_hbm,
             idx_s, idx_d, rows_a, rows_b, acc_sh,
             gs0, gs1, ss0, ss1, is0, is1):
        s = lax.axis_index("s")
        ones16 = jnp.ones((16,), jnp.float32)

        def fill(i, carry):
            for k in range(FEAT // 16):
                rows_a[i, pl.ds(k * 16, 16)] = ones16
            return carry

        lax.fori_loop(0, CHUNK, fill, 0)
        pltpu.sync_copy(
            z_hbm.at[pl.ds(s * ROWS_PER_TILE, ROWS_PER_TILE)],
            acc_sh.at[pl.ds(s * ROWS_PER_TILE, ROWS_PER_TILE)],
        )
        pltpu.sync_copy(dst_hbm.at[s, 0], idx_d.at[0])
        plsc.subcore_barrier()

        ssems = (ss0, ss1)
        isems = (is0, is1)

        def outer(b, carry):
            p = lax.rem(b, 2)
            q = lax.rem(b + 1, 2)

            @pl.when(b + 1 < nbatch)
            def _():
                pltpu.async_copy(dst_hbm.at[s, b + 1], idx_d.at[q], isems[1])

            @pl.when(b > 0)
            def _():
                pltpu.make_async_copy(dst_hbm.at[s, b], idx_d.at[p], isems[1]).wait()

            sc_desc = [None, None]
            for j in range(BCH):
                cur = j % 2
                if sc_desc[cur] is not None:
                    sc_desc[cur].wait()
                sc_desc[cur] = pltpu.async_copy(
                    rows_a, acc_sh.at[idx_d.at[p, j]], ssems[cur], add=True
                )
            for d in sc_desc:
                if d is not None:
                    d.wait()
            return carry

        lax.fori_loop(0, nbatch, outer, 0)
        plsc.subcore_barrier()
        pltpu.sync_copy(
            acc_sh.at[pl.ds(s * ROWS_PER_TILE, ROWS_PER_TILE)],
            out_hbm.at[pl.ds(s * ROWS_PER_TILE, ROWS_PER_TILE)],
        )

    return pl.kernel(
        body,
        out_type=jax.ShapeDtypeStruct((NPAD, FEAT), jnp.float32),
        mesh=_MESH,
        scratch_types=list(_SC_SCRATCH),
    )


_BR = 1264


def _dinv_block(deg_ref):
    return lax.rsqrt(jnp.maximum(deg_ref[:, :1], 1.0))


def _first_mm(x, w1, deg):
    def body(x_ref, w_ref, deg_ref, o_ref):
        dinv = _dinv_block(deg_ref)
        t = jnp.dot(x_ref[...], w_ref[...], preferred_element_type=jnp.float32)
        o_ref[...] = jnp.concatenate(
            [t * dinv, jnp.zeros((_BR, FEAT - HID), jnp.float32)], axis=1
        )

    return pl.pallas_call(
        body,
        grid=(NPAD // _BR,),
        in_specs=[
            pl.BlockSpec((_BR, FEAT), lambda i: (i, 0)),
            pl.BlockSpec((FEAT, HID), lambda i: (0, 0)),
            pl.BlockSpec((_BR, FEAT), lambda i: (i, 0)),
        ],
        out_specs=pl.BlockSpec((_BR, FEAT), lambda i: (i, 0)),
        out_shape=jax.ShapeDtypeStruct((NPAD, FEAT), jnp.float32),
    )(x, w1, deg)


def _mid_layer(agg, deg, b2d, wn):
    fo = wn.shape[1]

    def body(p_ref, deg_ref, b_ref, w_ref, o_ref):
        dinv = _dinv_block(deg_ref)
        u = jax.nn.relu(p_ref[:, :HID] * dinv + b_ref[0:1, :])
        t = jnp.dot(u, w_ref[...], preferred_element_type=jnp.float32) * dinv
        if fo == FEAT:
            o_ref[...] = t
        else:
            o_ref[...] = jnp.concatenate(
                [t, jnp.zeros((_BR, FEAT - fo), jnp.float32)], axis=1
            )

    return pl.pallas_call(
        body,
        grid=(NPAD // _BR,),
        in_specs=[
            pl.BlockSpec((_BR, FEAT), lambda i: (i, 0)),
            pl.BlockSpec((_BR, FEAT), lambda i: (i, 0)),
            pl.BlockSpec((8, HID), lambda i: (0, 0)),
            pl.BlockSpec((HID, fo), lambda i: (0, 0)),
        ],
        out_specs=pl.BlockSpec((_BR, FEAT), lambda i: (i, 0)),
        out_shape=jax.ShapeDtypeStruct((NPAD, FEAT), jnp.float32),
    )(agg, deg, b2d, wn)


def _hidden_layer(agg, deg, b2d, wa, ws):
    def body(p_ref, deg_ref, b_ref, wa_ref, ws_ref, o_ref):
        dinv = _dinv_block(deg_ref)
        u = jax.nn.relu(p_ref[:, :HID] * dinv + b_ref[0:1, :])
        ta = jnp.dot(u, wa_ref[...], preferred_element_type=jnp.float32)
        ts = jnp.dot(u, ws_ref[...], preferred_element_type=jnp.float32)
        o_ref[...] = jnp.concatenate([ta, ts], axis=1) * dinv

    return pl.pallas_call(
        body,
        grid=(NPAD // _BR,),
        in_specs=[
            pl.BlockSpec((_BR, FEAT), lambda i: (i, 0)),
            pl.BlockSpec((_BR, FEAT), lambda i: (i, 0)),
            pl.BlockSpec((8, HID), lambda i: (0, 0)),
            pl.BlockSpec((HID, HID), lambda i: (0, 0)),
            pl.BlockSpec((HID, HID), lambda i: (0, 0)),
        ],
        out_specs=pl.BlockSpec((_BR, FEAT), lambda i: (i, 0)),
        out_shape=jax.ShapeDtypeStruct((NPAD, FEAT), jnp.float32),
    )(agg, deg, b2d, wa, ws)


def _final_act(agg, deg, b2d, lo, width):
    def body(p_ref, deg_ref, b_ref, o_ref):
        dinv = _dinv_block(deg_ref)
        o_ref[...] = jax.nn.relu(
            p_ref[:, lo:lo + width] * dinv + b_ref[0:1, :]
        )

    return pl.pallas_call(
        body,
        grid=(NPAD // _BR,),
        in_specs=[
            pl.BlockSpec((_BR, FEAT), lambda i: (i, 0)),
            pl.BlockSpec((_BR, FEAT), lambda i: (i, 0)),
            pl.BlockSpec((8, width), lambda i: (0, 0)),
        ],
        out_specs=pl.BlockSpec((_BR, width), lambda i: (i, 0)),
        out_shape=jax.ShapeDtypeStruct((NPAD, width), jnp.float32),
    )(agg, deg, b2d)


_BS = 1024


def _struct_mm(s):
    def body(a_ref, b_ref, o_ref):
        o_ref[...] = lax.dot_general(
            a_ref[...], b_ref[...], (((1,), (1,)), ((), ())),
            preferred_element_type=jnp.float32,
        )

    nb = pl.cdiv(N, _BS)
    return pl.pallas_call(
        body,
        grid=(nb, nb),
        in_specs=[
            pl.BlockSpec((_BS, HID), lambda i, j: (i, 0)),
            pl.BlockSpec((_BS, HID), lambda i, j: (j, 0)),
        ],
        out_specs=pl.BlockSpec((_BS, _BS), lambda i, j: (i, j)),
        out_shape=jax.ShapeDtypeStruct((N, N), jnp.float32),
    )(s, s)


def kernel(x, edge_index, W1e, b1e, W2e, b2e, W1a, b1a, W2a, b2a, W1s, b1s):
    e = edge_index.shape[1]
    ea = e + N
    unit = NT * BCH * CHUNK
    ep = ((ea + unit - 1) // unit) * unit
    nbatch = ep // unit

    loop = jnp.arange(N, dtype=jnp.int32)
    src = jnp.concatenate([edge_index[0].astype(jnp.int32), loop])
    dst = jnp.concatenate([edge_index[1].astype(jnp.int32), loop])
    trash = N + (jnp.arange(ep, dtype=jnp.int32) % (NPAD - N))
    src_p = trash.at[:ea].set(src).reshape(NT, nbatch, BCH, CHUNK)
    dst_p = trash.at[:ea].set(dst).reshape(NT, nbatch, BCH, CHUNK)

    x_pad = jnp.zeros((NPAD, FEAT), jnp.float32).at[:N].set(x)
    b1e2 = jnp.broadcast_to(b1e, (8, HID))
    b2e2 = jnp.broadcast_to(b2e, (8, HID))
    b1a2 = jnp.broadcast_to(b1a, (8, HID))
    b2a2 = jnp.broadcast_to(b2a, (8, FEAT))
    b1s2 = jnp.broadcast_to(b1s, (8, HID))

    z128 = jnp.zeros((NPAD, FEAT), jnp.float32)
    agg0 = _agg_kernel(nbatch)
    agg = lambda h, sp, dp: agg0(h, sp, dp, z128)
    deg = _deg_kernel(nbatch)(dst_p, z128)

    t1 = _first_mm(x_pad, W1e, deg)
    p1 = agg(t1, src_p, dst_p)
    t2 = _mid_layer(p1, deg, b1e2, W2e)
    p2 = agg(t2, src_p, dst_p)
    t35 = _hidden_layer(p2, deg, b2e2, W1a, W1s)
    p35 = agg(t35, src_p, dst_p)
    t4 = _mid_layer(p35, deg, b1a2, W2a)
    p4 = agg(t4, src_p, dst_p)
    x_hat = _final_act(p4, deg, b2a2, 0, FEAT)
    s = _final_act(p35, deg, b1s2, HID, HID)

    struct = _struct_mm(s)
    return (struct, x_hat[:N])
